# baseline (device time: 56361 ns/iter reference)
import jax
import jax.numpy as jnp
from jax import lax
from jax.experimental import pallas as pl
from jax.experimental.pallas import tpu as pltpu

N_DEV = 8
N_HOP = N_DEV - 1
NS = 4


def kernel(A, B):
    m, _ = A.shape
    _, n = B.shape
    chunk = m // N_DEV
    half = chunk // 2
    sub = half // NS

    def body(a_ref, b_ref, out_ref, c_r, c_l, b16,
             send_r, recv_r, send_l, recv_l):
        d = lax.axis_index("i")
        left = lax.rem(d + N_DEV - 1, N_DEV)
        right = lax.rem(d + 1, N_DEV)

        b16[...] = b_ref[...].astype(jnp.bfloat16)

        def dot_rows(row0):
            return jnp.dot(
                a_ref[pl.ds(row0, half), :].astype(jnp.bfloat16), b16[...],
                preferred_element_type=jnp.float32,
            )

        def stream(q):
            k = q // 2
            if q % 2 == 0:
                return c_r, send_r, recv_r, right, k * sub, k
            return c_l, send_l, recv_l, left, half + k * sub, k

        rdmas = {}

        def start_hop(q, s):
            comm, ssem, rsem, nbr, _, k = stream(q)
            src = N_HOP if s == 0 else s - 1
            rd = pltpu.make_async_remote_copy(
                src_ref=comm.at[k, src], dst_ref=comm.at[k, s],
                send_sem=ssem.at[k, s], recv_sem=rsem.at[k, s],
                device_id=(nbr,), device_id_type=pl.DeviceIdType.MESH,
            )
            rdmas[(q, s)] = rd
            rd.start()

        pr = dot_rows(lax.rem(d + N_DEV - 1, N_DEV) * chunk)
        pl_ = dot_rows(lax.rem(d + 1, N_DEV) * chunk + half)
        for k in range(NS):
            c_r[k, N_HOP] = pr[k * sub:(k + 1) * sub].astype(jnp.bfloat16)
            c_l[k, N_HOP] = pl_[k * sub:(k + 1) * sub].astype(jnp.bfloat16)

        barrier_sem = pltpu.get_barrier_semaphore()
        for nbr in (left, right):
            pl.semaphore_signal(
                barrier_sem, inc=1,
                device_id=(nbr,), device_id_type=pl.DeviceIdType.MESH,
            )
        pl.semaphore_wait(barrier_sem, 2)

        for q in range(2 * NS):
            start_hop(q, 0)

        for s in range(N_HOP):
            cr = lax.rem(d + 2 * N_DEV - s - 2, N_DEV)
            cl = lax.rem(d + s + 2, N_DEV)
            pr = dot_rows(cr * chunk)
            pl_ = dot_rows(cl * chunk + half)
            for q in range(2 * NS):
                comm, _, _, _, out_row, k = stream(q)
                part = (pr if q % 2 == 0 else pl_)[k * sub:(k + 1) * sub]
                rdmas[(q, s)].wait_recv()
                acc = comm[k, s].astype(jnp.float32) + part
                if s < N_HOP - 1:
                    comm[k, s] = acc.astype(jnp.bfloat16)
                    start_hop(q, s + 1)
                else:
                    out_ref[pl.ds(out_row, sub), :] = acc

        for q in range(2 * NS):
            for s in range(N_HOP):
                rdmas[(q, s)].wait_send()

    comm_shape = pltpu.VMEM((NS, N_DEV, sub, n), jnp.bfloat16)
    dir_sems = pltpu.SemaphoreType.DMA((NS, N_HOP))
    return pl.pallas_call(
        body,
        out_shape=jax.ShapeDtypeStruct((chunk, n), jnp.float32),
        in_specs=[
            pl.BlockSpec(memory_space=pltpu.VMEM),
            pl.BlockSpec(memory_space=pltpu.VMEM),
        ],
        out_specs=pl.BlockSpec(memory_space=pltpu.VMEM),
        scratch_shapes=[
            comm_shape, comm_shape,
            pltpu.VMEM((B.shape[0], n), jnp.bfloat16),
            dir_sems, dir_sems,
            dir_sems, dir_sems,
        ],
        compiler_params=pltpu.CompilerParams(collective_id=0),
    )(A, B)
